# Initial kernel scaffold; baseline (speedup 1.0000x reference)
import jax
import jax.numpy as jnp
from jax.experimental import pallas as pl


def kernel(x, gate_w, ln_g, ln_b, W1, b1, W2, b2):
    raise NotImplementedError("write your pallas kernel here")


# speedup vs baseline: 16.4130x; 16.4130x over previous
import functools

import jax
import jax.numpy as jnp
from jax import lax
from jax.experimental import pallas as pl
from jax.experimental.pallas import tpu as pltpu
from jax.experimental.pallas import tpu_sc as plsc

B, L, D, H, NC = 2, 2048, 768, 3072, 64
E = NC + 1
N = B * L
T = 128
NT = N // T
G = NT + E
TR = 512
EP = 128


def _router_body(x_ref, gw_ref, g_ref, b_ref, xn_ref, eid_ref):
    x = x_ref[...]
    m = jnp.mean(x, axis=1, keepdims=True)
    xc = x - m
    v = jnp.mean(xc * xc, axis=1, keepdims=True)
    xn = xc / jnp.sqrt(v + 1e-5) * g_ref[...] + b_ref[...]
    xn_ref[...] = xn
    logits = lax.dot_general(xn, gw_ref[...], (((1,), (1,)), ((), ())),
                             preferred_element_type=jnp.float32)
    col = lax.broadcasted_iota(jnp.int32, (TR, EP), 1)
    logits = jnp.where(col < E, logits, -jnp.inf)
    eid_ref[...] = jnp.argmax(logits, axis=1).astype(jnp.int32).reshape(TR, 1)


def _router(xf, gate_w, ln_g, ln_b):
    gw = jnp.zeros((EP, D), jnp.float32).at[:E].set(gate_w)
    return pl.pallas_call(
        _router_body,
        grid=(N // TR,),
        in_specs=[
            pl.BlockSpec((TR, D), lambda i: (i, 0)),
            pl.BlockSpec((EP, D), lambda i: (0, 0)),
            pl.BlockSpec((1, D), lambda i: (0, 0)),
            pl.BlockSpec((1, D), lambda i: (0, 0)),
        ],
        out_specs=[
            pl.BlockSpec((TR, D), lambda i: (i, 0)),
            pl.BlockSpec((TR, 1), lambda i: (i, 0)),
        ],
        out_shape=[
            jax.ShapeDtypeStruct((N, D), jnp.float32),
            jax.ShapeDtypeStruct((N, 1), jnp.int32),
        ],
    )(xf, gw, ln_g.reshape(1, D), ln_b.reshape(1, D))


_NW = 32
_RW = N // _NW


def _sc_mesh():
    return plsc.VectorSubcoreMesh(core_axis_name="c", subcore_axis_name="s")


def _gather2(t1, t2, idx):
    ch = _RW // 2

    @functools.partial(
        pl.kernel,
        out_type=[jax.ShapeDtypeStruct((N, D), jnp.float32),
                  jax.ShapeDtypeStruct((N, D), jnp.float32)],
        mesh=_sc_mesh(),
        scratch_types=[
            pltpu.VMEM((ch,), jnp.int32),
            pltpu.VMEM((ch, D), jnp.float32),
            pltpu.VMEM((ch, D), jnp.float32),
            pltpu.SemaphoreType.DMA,
            pltpu.SemaphoreType.DMA,
        ],
    )
    def k(t1_hbm, t2_hbm, idx_hbm, o1_hbm, o2_hbm, idx_v, r1_v, r2_v, s1, s2):
        wid = lax.axis_index("s") * 2 + lax.axis_index("c")
        for c in range(_RW // ch):
            base = wid * _RW + c * ch
            pltpu.sync_copy(idx_hbm.at[pl.ds(base, ch)], idx_v)
            cp1 = pltpu.async_copy(t1_hbm.at[idx_v], r1_v, s1)
            cp2 = pltpu.async_copy(t2_hbm.at[idx_v], r2_v, s2)
            cp1.wait()
            pltpu.sync_copy(r1_v, o1_hbm.at[pl.ds(base, ch)])
            cp2.wait()
            pltpu.sync_copy(r2_v, o2_hbm.at[pl.ds(base, ch)])

    return k(t1, t2, idx)


def _gather1(t1, idx):
    @functools.partial(
        pl.kernel,
        out_type=jax.ShapeDtypeStruct((N, D), jnp.float32),
        mesh=_sc_mesh(),
        scratch_types=[
            pltpu.VMEM((_RW,), jnp.int32),
            pltpu.VMEM((_RW, D), jnp.float32),
            pltpu.SemaphoreType.DMA,
        ],
    )
    def k(t1_hbm, idx_hbm, o1_hbm, idx_v, rows_v, sem):
        wid = lax.axis_index("s") * 2 + lax.axis_index("c")
        base = wid * _RW
        pltpu.sync_copy(idx_hbm.at[pl.ds(base, _RW)], idx_v)
        pltpu.async_copy(t1_hbm.at[idx_v], rows_v, sem).wait()
        pltpu.sync_copy(rows_v, o1_hbm.at[pl.ds(base, _RW)])

    return k(t1, idx)


def _moe_body(tile_r, exp_r, start_r, end_r, first_r,
              xs_ref, xr_ref, w1_ref, b1_ref, w2_ref, b2_ref, out_ref):
    g = pl.program_id(0)

    @pl.when(first_r[g] == 1)
    def _():
        out_ref[...] = xr_ref[...]

    @pl.when(end_r[g] > start_r[g])
    def _():
        x = xs_ref[...]
        h = lax.dot_general(x, w1_ref[0], (((1,), (1,)), ((), ())),
                            preferred_element_type=jnp.float32) + b1_ref[0]
        h = 0.5 * h * (1.0 + lax.erf(h * (2.0 ** -0.5)))
        y = lax.dot_general(h, w2_ref[0], (((1,), (1,)), ((), ())),
                            preferred_element_type=jnp.float32) + b2_ref[0]
        lo = start_r[g] - tile_r[g] * T
        hi = end_r[g] - tile_r[g] * T
        row = lax.broadcasted_iota(jnp.int32, (T, 1), 0)
        mask = (row >= lo) & (row < hi)
        out_ref[...] += jnp.where(mask, y, 0.0)


def _grouped_ffn(xs, xr, W1, b1, W2, b2, tile_a, exp_a, start_a, end_a, first_a):
    grid_spec = pltpu.PrefetchScalarGridSpec(
        num_scalar_prefetch=5,
        grid=(G,),
        in_specs=[
            pl.BlockSpec((T, D), lambda g, t, e, s, en, f: (t[g], 0)),
            pl.BlockSpec((T, D), lambda g, t, e, s, en, f: (t[g], 0)),
            pl.BlockSpec((1, H, D), lambda g, t, e, s, en, f: (e[g], 0, 0)),
            pl.BlockSpec((1, 1, H), lambda g, t, e, s, en, f: (e[g], 0, 0)),
            pl.BlockSpec((1, D, H), lambda g, t, e, s, en, f: (e[g], 0, 0)),
            pl.BlockSpec((1, 1, D), lambda g, t, e, s, en, f: (e[g], 0, 0)),
        ],
        out_specs=pl.BlockSpec((T, D), lambda g, t, e, s, en, f: (t[g], 0)),
    )
    return pl.pallas_call(
        _moe_body,
        grid_spec=grid_spec,
        out_shape=jax.ShapeDtypeStruct((N, D), jnp.float32),
        compiler_params=pltpu.CompilerParams(
            dimension_semantics=("arbitrary",),
        ),
    )(tile_a, exp_a, start_a, end_a, first_a, xs, xr,
      W1, b1.reshape(E, 1, H), W2, b2.reshape(E, 1, D))


def _make_items(eid):
    perm = jnp.argsort(eid)
    seid = eid[perm]
    pos = jnp.arange(N, dtype=jnp.int32)
    start_flag = (pos % T == 0) | (seid != jnp.roll(seid, 1))
    p, = jnp.nonzero(start_flag, size=G, fill_value=0)
    p = p.astype(jnp.int32)
    num_items = jnp.sum(start_flag.astype(jnp.int32))
    gi = jnp.arange(G, dtype=jnp.int32)
    valid = gi < num_items
    p_last = jnp.max(jnp.where(valid, p, 0))
    p_eff = jnp.where(valid, p, p_last)
    p_shift = jnp.concatenate([p[1:], jnp.zeros((1,), jnp.int32)])
    end = jnp.where(gi == num_items - 1, N, p_shift)
    end = jnp.where(valid, end, p_eff)
    tile_a = p_eff // T
    exp_a = seid[p_eff]
    first_a = ((p_eff % T == 0) & valid).astype(jnp.int32)
    inv = jnp.zeros((N,), jnp.int32).at[perm].set(pos)
    return perm.astype(jnp.int32), inv, tile_a, exp_a, p_eff, end, first_a


def kernel(x, gate_w, ln_g, ln_b, W1, b1, W2, b2):
    xf = x.reshape(N, D)
    xn, eid2 = _router(xf, gate_w, ln_g, ln_b)
    eid = eid2.reshape(N)
    perm, inv, tile_a, exp_a, start_a, end_a, first_a = _make_items(eid)
    xs, xr = _gather2(xn, xf, perm)
    ys = _grouped_ffn(xs, xr, W1, b1, W2, b2,
                      tile_a, exp_a, start_a, end_a, first_a)
    out = _gather1(ys, inv)
    return out.reshape(B, L, D)


# speedup vs baseline: 16.6101x; 1.0120x over previous
import functools

import jax
import jax.numpy as jnp
from jax import lax
from jax.experimental import pallas as pl
from jax.experimental.pallas import tpu as pltpu
from jax.experimental.pallas import tpu_sc as plsc

B, L, D, H, NC = 2, 2048, 768, 3072, 64
E = NC + 1
N = B * L
T = 128
NT = N // T
G = NT + E
TR = 512
EP = 128


def _router_body(x_ref, gw_ref, g_ref, b_ref, xn_ref, eid_ref):
    x = x_ref[...]
    m = jnp.mean(x, axis=1, keepdims=True)
    xc = x - m
    v = jnp.mean(xc * xc, axis=1, keepdims=True)
    xn = xc / jnp.sqrt(v + 1e-5) * g_ref[...] + b_ref[...]
    xn_ref[...] = xn
    logits = lax.dot_general(xn, gw_ref[...], (((1,), (1,)), ((), ())),
                             preferred_element_type=jnp.float32)
    col = lax.broadcasted_iota(jnp.int32, (TR, EP), 1)
    logits = jnp.where(col < E, logits, -jnp.inf)
    eid_ref[...] = jnp.argmax(logits, axis=1).astype(jnp.int32).reshape(TR, 1)


def _router(xf, gate_w, ln_g, ln_b):
    gw = jnp.zeros((EP, D), jnp.float32).at[:E].set(gate_w)
    return pl.pallas_call(
        _router_body,
        grid=(N // TR,),
        in_specs=[
            pl.BlockSpec((TR, D), lambda i: (i, 0)),
            pl.BlockSpec((EP, D), lambda i: (0, 0)),
            pl.BlockSpec((1, D), lambda i: (0, 0)),
            pl.BlockSpec((1, D), lambda i: (0, 0)),
        ],
        out_specs=[
            pl.BlockSpec((TR, D), lambda i: (i, 0)),
            pl.BlockSpec((TR, 1), lambda i: (i, 0)),
        ],
        out_shape=[
            jax.ShapeDtypeStruct((N, D), jnp.float32),
            jax.ShapeDtypeStruct((N, 1), jnp.int32),
        ],
    )(xf, gw, ln_g.reshape(1, D), ln_b.reshape(1, D))


_NW = 32
_RW = N // _NW


def _sc_mesh():
    return plsc.VectorSubcoreMesh(core_axis_name="c", subcore_axis_name="s")


def _gather2(t1, t2, idx):
    ch = _RW // 2

    @functools.partial(
        pl.kernel,
        out_type=[jax.ShapeDtypeStruct((N, D), jnp.float32),
                  jax.ShapeDtypeStruct((N, D), jnp.float32)],
        mesh=_sc_mesh(),
        scratch_types=[
            pltpu.VMEM((ch,), jnp.int32),
            pltpu.VMEM((ch, D), jnp.float32),
            pltpu.VMEM((ch, D), jnp.float32),
            pltpu.SemaphoreType.DMA,
            pltpu.SemaphoreType.DMA,
        ],
    )
    def k(t1_hbm, t2_hbm, idx_hbm, o1_hbm, o2_hbm, idx_v, r1_v, r2_v, s1, s2):
        wid = lax.axis_index("s") * 2 + lax.axis_index("c")
        for c in range(_RW // ch):
            base = wid * _RW + c * ch
            pltpu.sync_copy(idx_hbm.at[pl.ds(base, ch)], idx_v)
            cp1 = pltpu.async_copy(t1_hbm.at[idx_v], r1_v, s1)
            cp2 = pltpu.async_copy(t2_hbm.at[idx_v], r2_v, s2)
            cp1.wait()
            pltpu.sync_copy(r1_v, o1_hbm.at[pl.ds(base, ch)])
            cp2.wait()
            pltpu.sync_copy(r2_v, o2_hbm.at[pl.ds(base, ch)])

    return k(t1, t2, idx)


def _gather1(t1, idx):
    @functools.partial(
        pl.kernel,
        out_type=jax.ShapeDtypeStruct((N, D), jnp.float32),
        mesh=_sc_mesh(),
        scratch_types=[
            pltpu.VMEM((_RW,), jnp.int32),
            pltpu.VMEM((_RW, D), jnp.float32),
            pltpu.SemaphoreType.DMA,
        ],
    )
    def k(t1_hbm, idx_hbm, o1_hbm, idx_v, rows_v, sem):
        wid = lax.axis_index("s") * 2 + lax.axis_index("c")
        base = wid * _RW
        pltpu.sync_copy(idx_hbm.at[pl.ds(base, _RW)], idx_v)
        pltpu.async_copy(t1_hbm.at[idx_v], rows_v, sem).wait()
        pltpu.sync_copy(rows_v, o1_hbm.at[pl.ds(base, _RW)])

    return k(t1, idx)


QS = 4
HQ = H // QS


def _moe_body(tile_r, exp_r, start_r, end_r, first_r,
              xs_ref, xr_ref, *rest):
    w1_refs = rest[0:QS]
    b1_refs = rest[QS:2 * QS]
    w2_refs = rest[2 * QS:3 * QS]
    b2_ref = rest[3 * QS]
    out_ref = rest[3 * QS + 1]
    g = pl.program_id(0)

    @pl.when(first_r[g] == 1)
    def _():
        out_ref[...] = xr_ref[...]

    @pl.when(end_r[g] > start_r[g])
    def _():
        x = xs_ref[...]
        y = jnp.broadcast_to(b2_ref[0], (T, D))
        for q in range(QS):
            h = lax.dot_general(x, w1_refs[q][0], (((1,), (1,)), ((), ())),
                                preferred_element_type=jnp.float32) + b1_refs[q][0]
            h = 0.5 * h * (1.0 + lax.erf(h * (2.0 ** -0.5)))
            y = y + lax.dot_general(h, w2_refs[q][0], (((1,), (1,)), ((), ())),
                                    preferred_element_type=jnp.float32)
        lo = start_r[g] - tile_r[g] * T
        hi = end_r[g] - tile_r[g] * T
        row = lax.broadcasted_iota(jnp.int32, (T, 1), 0)
        mask = (row >= lo) & (row < hi)
        out_ref[...] += jnp.where(mask, y, 0.0)


def _grouped_ffn(xs, xr, W1, b1, W2, b2, tile_a, exp_a, start_a, end_a, first_a):
    def w1_spec(q):
        return pl.BlockSpec((1, HQ, D), lambda g, t, e, s, en, f: (e[g], q, 0))

    def b1_spec(q):
        return pl.BlockSpec((1, 1, HQ), lambda g, t, e, s, en, f: (e[g], 0, q))

    def w2_spec(q):
        return pl.BlockSpec((1, D, HQ), lambda g, t, e, s, en, f: (e[g], 0, q))

    grid_spec = pltpu.PrefetchScalarGridSpec(
        num_scalar_prefetch=5,
        grid=(G,),
        in_specs=[
            pl.BlockSpec((T, D), lambda g, t, e, s, en, f: (t[g], 0)),
            pl.BlockSpec((T, D), lambda g, t, e, s, en, f: (t[g], 0)),
            *[w1_spec(q) for q in range(QS)],
            *[b1_spec(q) for q in range(QS)],
            *[w2_spec(q) for q in range(QS)],
            pl.BlockSpec((1, 1, D), lambda g, t, e, s, en, f: (e[g], 0, 0)),
        ],
        out_specs=pl.BlockSpec((T, D), lambda g, t, e, s, en, f: (t[g], 0)),
    )
    return pl.pallas_call(
        _moe_body,
        grid_spec=grid_spec,
        out_shape=jax.ShapeDtypeStruct((N, D), jnp.float32),
        compiler_params=pltpu.CompilerParams(
            dimension_semantics=("arbitrary",),
        ),
    )(tile_a, exp_a, start_a, end_a, first_a, xs, xr,
      *([W1] * QS), *([b1.reshape(E, 1, H)] * QS), *([W2] * QS),
      b2.reshape(E, 1, D))


def _make_items(eid):
    perm = jnp.argsort(eid)
    seid = eid[perm]
    pos = jnp.arange(N, dtype=jnp.int32)
    start_flag = (pos % T == 0) | (seid != jnp.roll(seid, 1))
    p, = jnp.nonzero(start_flag, size=G, fill_value=0)
    p = p.astype(jnp.int32)
    num_items = jnp.sum(start_flag.astype(jnp.int32))
    gi = jnp.arange(G, dtype=jnp.int32)
    valid = gi < num_items
    p_last = jnp.max(jnp.where(valid, p, 0))
    p_eff = jnp.where(valid, p, p_last)
    p_shift = jnp.concatenate([p[1:], jnp.zeros((1,), jnp.int32)])
    end = jnp.where(gi == num_items - 1, N, p_shift)
    end = jnp.where(valid, end, p_eff)
    tile_a = p_eff // T
    exp_a = seid[p_eff]
    first_a = ((p_eff % T == 0) & valid).astype(jnp.int32)
    inv = jnp.zeros((N,), jnp.int32).at[perm].set(pos)
    return perm.astype(jnp.int32), inv, tile_a, exp_a, p_eff, end, first_a


def kernel(x, gate_w, ln_g, ln_b, W1, b1, W2, b2):
    xf = x.reshape(N, D)
    xn, eid2 = _router(xf, gate_w, ln_g, ln_b)
    eid = eid2.reshape(N)
    perm, inv, tile_a, exp_a, start_a, end_a, first_a = _make_items(eid)
    xs, xr = _gather2(xn, xf, perm)
    ys = _grouped_ffn(xs, xr, W1, b1, W2, b2,
                      tile_a, exp_a, start_a, end_a, first_a)
    out = _gather1(ys, inv)
    return out.reshape(B, L, D)


# speedup vs baseline: 16.7325x; 1.0074x over previous
import functools

import jax
import jax.numpy as jnp
from jax import lax
from jax.experimental import pallas as pl
from jax.experimental.pallas import tpu as pltpu
from jax.experimental.pallas import tpu_sc as plsc

B, L, D, H, NC = 2, 2048, 768, 3072, 64
E = NC + 1
N = B * L
T = 128
NT = N // T
G = NT + E
TR = 512
EP = 128


def _router_body(x_ref, gw_ref, g_ref, b_ref, xn_ref, eid_ref):
    x = x_ref[...]
    m = jnp.mean(x, axis=1, keepdims=True)
    xc = x - m
    v = jnp.mean(xc * xc, axis=1, keepdims=True)
    xn = xc / jnp.sqrt(v + 1e-5) * g_ref[...] + b_ref[...]
    xn_ref[...] = xn
    logits = lax.dot_general(xn, gw_ref[...], (((1,), (1,)), ((), ())),
                             preferred_element_type=jnp.float32)
    col = lax.broadcasted_iota(jnp.int32, (TR, EP), 1)
    logits = jnp.where(col < E, logits, -jnp.inf)
    eid_ref[...] = jnp.argmax(logits, axis=1).astype(jnp.int32).reshape(TR, 1)


def _router(xf, gate_w, ln_g, ln_b):
    gw = jnp.zeros((EP, D), jnp.float32).at[:E].set(gate_w)
    return pl.pallas_call(
        _router_body,
        grid=(N // TR,),
        in_specs=[
            pl.BlockSpec((TR, D), lambda i: (i, 0)),
            pl.BlockSpec((EP, D), lambda i: (0, 0)),
            pl.BlockSpec((1, D), lambda i: (0, 0)),
            pl.BlockSpec((1, D), lambda i: (0, 0)),
        ],
        out_specs=[
            pl.BlockSpec((TR, D), lambda i: (i, 0)),
            pl.BlockSpec((TR, 1), lambda i: (i, 0)),
        ],
        out_shape=[
            jax.ShapeDtypeStruct((N, D), jnp.float32),
            jax.ShapeDtypeStruct((N, 1), jnp.int32),
        ],
    )(xf, gw, ln_g.reshape(1, D), ln_b.reshape(1, D))


_NW = 32
_RW = N // _NW


def _sc_mesh():
    return plsc.VectorSubcoreMesh(core_axis_name="c", subcore_axis_name="s")


def _gather2(t1, t2, idx):
    ch = _RW // 2

    @functools.partial(
        pl.kernel,
        out_type=[jax.ShapeDtypeStruct((N, D), jnp.float32),
                  jax.ShapeDtypeStruct((N, D), jnp.float32)],
        mesh=_sc_mesh(),
        scratch_types=[
            pltpu.VMEM((ch,), jnp.int32),
            pltpu.VMEM((ch, D), jnp.float32),
            pltpu.VMEM((ch, D), jnp.float32),
            pltpu.SemaphoreType.DMA,
            pltpu.SemaphoreType.DMA,
        ],
    )
    def k(t1_hbm, t2_hbm, idx_hbm, o1_hbm, o2_hbm, idx_v, r1_v, r2_v, s1, s2):
        wid = lax.axis_index("s") * 2 + lax.axis_index("c")
        for c in range(_RW // ch):
            base = wid * _RW + c * ch
            pltpu.sync_copy(idx_hbm.at[pl.ds(base, ch)], idx_v)
            cp1 = pltpu.async_copy(t1_hbm.at[idx_v], r1_v, s1)
            cp2 = pltpu.async_copy(t2_hbm.at[idx_v], r2_v, s2)
            cp1.wait()
            pltpu.sync_copy(r1_v, o1_hbm.at[pl.ds(base, ch)])
            cp2.wait()
            pltpu.sync_copy(r2_v, o2_hbm.at[pl.ds(base, ch)])

    return k(t1, t2, idx)


def _gather1(t1, idx):
    @functools.partial(
        pl.kernel,
        out_type=jax.ShapeDtypeStruct((N, D), jnp.float32),
        mesh=_sc_mesh(),
        scratch_types=[
            pltpu.VMEM((_RW,), jnp.int32),
            pltpu.VMEM((_RW, D), jnp.float32),
            pltpu.SemaphoreType.DMA,
        ],
    )
    def k(t1_hbm, idx_hbm, o1_hbm, idx_v, rows_v, sem):
        wid = lax.axis_index("s") * 2 + lax.axis_index("c")
        base = wid * _RW
        pltpu.sync_copy(idx_hbm.at[pl.ds(base, _RW)], idx_v)
        pltpu.async_copy(t1_hbm.at[idx_v], rows_v, sem).wait()
        pltpu.sync_copy(rows_v, o1_hbm.at[pl.ds(base, _RW)])

    return k(t1, idx)


QS = 6
HQ = H // QS
DQ = D // QS


def _moe_body(tile_r, exp_r, start_r, end_r, first_r,
              xs_ref, xr_ref, *rest):
    w1_refs = rest[0:QS]
    b1_refs = rest[QS:2 * QS]
    w2_refs = rest[2 * QS:3 * QS]
    b2_ref = rest[3 * QS]
    out_ref = rest[3 * QS + 1]
    g = pl.program_id(0)

    @pl.when(first_r[g] == 1)
    def _():
        out_ref[...] = xr_ref[...]

    @pl.when(end_r[g] > start_r[g])
    def _():
        x = xs_ref[...]
        hs = []
        for q in range(QS):
            h = lax.dot_general(x, w1_refs[q][0], (((1,), (1,)), ((), ())),
                                preferred_element_type=jnp.float32) + b1_refs[q][0]
            hs.append(0.5 * h * (1.0 + lax.erf(h * (2.0 ** -0.5))))
        h = jnp.concatenate(hs, axis=1)
        lo = start_r[g] - tile_r[g] * T
        hi = end_r[g] - tile_r[g] * T
        row = lax.broadcasted_iota(jnp.int32, (T, 1), 0)
        mask = (row >= lo) & (row < hi)
        for q in range(QS):
            y = lax.dot_general(h, w2_refs[q][0], (((1,), (1,)), ((), ())),
                                preferred_element_type=jnp.float32)
            y = y + b2_ref[0, :, pl.ds(q * DQ, DQ)]
            out_ref[:, pl.ds(q * DQ, DQ)] += jnp.where(mask, y, 0.0)


def _grouped_ffn(xs, xr, W1, b1, W2, b2, tile_a, exp_a, start_a, end_a, first_a):
    def w1_spec(q):
        return pl.BlockSpec((1, HQ, D), lambda g, t, e, s, en, f: (e[g], q, 0))

    def b1_spec(q):
        return pl.BlockSpec((1, 1, HQ), lambda g, t, e, s, en, f: (e[g], 0, q))

    def w2_spec(q):
        return pl.BlockSpec((1, DQ, H), lambda g, t, e, s, en, f: (e[g], q, 0))

    grid_spec = pltpu.PrefetchScalarGridSpec(
        num_scalar_prefetch=5,
        grid=(G,),
        in_specs=[
            pl.BlockSpec((T, D), lambda g, t, e, s, en, f: (t[g], 0)),
            pl.BlockSpec((T, D), lambda g, t, e, s, en, f: (t[g], 0)),
            *[w1_spec(q) for q in range(QS)],
            *[b1_spec(q) for q in range(QS)],
            *[w2_spec(q) for q in range(QS)],
            pl.BlockSpec((1, 1, D), lambda g, t, e, s, en, f: (e[g], 0, 0)),
        ],
        out_specs=pl.BlockSpec((T, D), lambda g, t, e, s, en, f: (t[g], 0)),
    )
    return pl.pallas_call(
        _moe_body,
        grid_spec=grid_spec,
        out_shape=jax.ShapeDtypeStruct((N, D), jnp.float32),
        compiler_params=pltpu.CompilerParams(
            dimension_semantics=("arbitrary",),
        ),
    )(tile_a, exp_a, start_a, end_a, first_a, xs, xr,
      *([W1] * QS), *([b1.reshape(E, 1, H)] * QS), *([W2] * QS),
      b2.reshape(E, 1, D))


def _make_items(eid):
    perm = jnp.argsort(eid)
    seid = eid[perm]
    pos = jnp.arange(N, dtype=jnp.int32)
    start_flag = (pos % T == 0) | (seid != jnp.roll(seid, 1))
    p, = jnp.nonzero(start_flag, size=G, fill_value=0)
    p = p.astype(jnp.int32)
    num_items = jnp.sum(start_flag.astype(jnp.int32))
    gi = jnp.arange(G, dtype=jnp.int32)
    valid = gi < num_items
    p_last = jnp.max(jnp.where(valid, p, 0))
    p_eff = jnp.where(valid, p, p_last)
    p_shift = jnp.concatenate([p[1:], jnp.zeros((1,), jnp.int32)])
    end = jnp.where(gi == num_items - 1, N, p_shift)
    end = jnp.where(valid, end, p_eff)
    tile_a = p_eff // T
    exp_a = seid[p_eff]
    first_a = ((p_eff % T == 0) & valid).astype(jnp.int32)
    inv = jnp.zeros((N,), jnp.int32).at[perm].set(pos)
    return perm.astype(jnp.int32), inv, tile_a, exp_a, p_eff, end, first_a


def kernel(x, gate_w, ln_g, ln_b, W1, b1, W2, b2):
    xf = x.reshape(N, D)
    xn, eid2 = _router(xf, gate_w, ln_g, ln_b)
    eid = eid2.reshape(N)
    perm, inv, tile_a, exp_a, start_a, end_a, first_a = _make_items(eid)
    xs, xr = _gather2(xn, xf, perm)
    ys = _grouped_ffn(xs, xr, W1, b1, W2, b2,
                      tile_a, exp_a, start_a, end_a, first_a)
    out = _gather1(ys, inv)
    return out.reshape(B, L, D)
